# Initial kernel scaffold; baseline (speedup 1.0000x reference)
import jax
import jax.numpy as jnp
from jax.experimental import pallas as pl


def kernel(x, edge_index, W1, b1, W2, b2, Wfc, bfc):
    raise NotImplementedError("write your pallas kernel here")


# speedup vs baseline: 5.4256x; 5.4256x over previous
import functools

import jax
import jax.numpy as jnp
from jax import lax
from jax.experimental import pallas as pl
from jax.experimental.pallas import tpu as pltpu
from jax.experimental.pallas import tpu_sc as plsc

N = 10000
NP = 10240
E = 160000
EP = 163840
D_IN = 256
D_HID = 512
NQ = 4
NC = 2
NS = 16
C = 128
RT = NP // NS
LCH = EP // NS // C
DCH = EP // (NC * NS) // C

_mesh = plsc.VectorSubcoreMesh(core_axis_name="c", subcore_axis_name="s")


@functools.partial(
    pl.kernel,
    out_type=jax.ShapeDtypeStruct((NC, NP, 16), jnp.float32),
    mesh=_mesh,
    scratch_types=[
        pltpu.VMEM((DCH, C), jnp.int32),
        pltpu.VMEM((C, 16), jnp.float32),
        pltpu.VMEM_SHARED((NP, 16), jnp.float32),
    ],
)
def _deg_counts(dst_hbm, ones_hbm, zeros_hbm, out_hbm, idx_v, ones_v, acc_sh):
    c = lax.axis_index("c")
    s = lax.axis_index("s")
    pltpu.sync_copy(dst_hbm.at[c, s], idx_v)
    pltpu.sync_copy(ones_hbm, ones_v)
    pltpu.sync_copy(zeros_hbm, acc_sh.at[pl.ds(s * RT, RT)])
    plsc.subcore_barrier()

    def body(j, carry):
        pltpu.sync_copy(ones_v, acc_sh.at[idx_v.at[j]], add=True)
        return carry

    lax.fori_loop(0, DCH, body, 0)
    plsc.subcore_barrier()
    pltpu.sync_copy(acc_sh.at[pl.ds(s * RT, RT)], out_hbm.at[c, pl.ds(s * RT, RT)])


@functools.partial(
    pl.kernel,
    out_type=jax.ShapeDtypeStruct((NQ, NP, 128), jnp.float32),
    mesh=_mesh,
    scratch_types=[
        pltpu.VMEM((LCH, C), jnp.int32),
        pltpu.VMEM((LCH, C), jnp.int32),
        pltpu.VMEM((C,), jnp.int32),
        pltpu.VMEM((C, 128), jnp.float32),
        pltpu.VMEM_SHARED((NP, 128), jnp.float32),
        pltpu.SemaphoreType.DMA,
    ],
)
def _gcn_aggregate(g_hbm, src_hbm, dst_hbm, out_hbm,
                   src_v, dst_v, idx_v, rows_v, acc_sh, sem):
    c = lax.axis_index("c")
    s = lax.axis_index("s")
    pltpu.sync_copy(src_hbm.at[s], src_v)
    pltpu.sync_copy(dst_hbm.at[s], dst_v)
    for qi in range(NQ // NC):
        q = c * (NQ // NC) + qi
        row0 = q * NP
        pltpu.sync_copy(g_hbm.at[pl.ds(row0 + s * RT, RT)],
                        acc_sh.at[pl.ds(s * RT, RT)])
        plsc.subcore_barrier()

        def body(j, carry):
            for k in range(C // 16):
                sl = pl.ds(k * 16, 16)
                idx_v[sl] = src_v[j, sl] + row0
            pltpu.async_copy(g_hbm.at[idx_v], rows_v, sem).wait()
            pltpu.sync_copy(rows_v, acc_sh.at[dst_v.at[j]], add=True)
            return carry

        lax.fori_loop(0, LCH, body, 0)
        plsc.subcore_barrier()
        pltpu.sync_copy(acc_sh.at[pl.ds(s * RT, RT)],
                        out_hbm.at[q, pl.ds(s * RT, RT)])


def _dis_from(degp):
    deg = 1.0 + degp[0, :, 0:1] + degp[1, :, 0:1]
    return lax.rsqrt(deg)


def _tc1_body(x_ref, w_ref, degp_ref, out_ref):
    dis = _dis_from(degp_ref[...])
    out_ref[0] = jnp.dot(x_ref[...], w_ref[...],
                         preferred_element_type=jnp.float32) * dis


def _tc2_body(s_ref, degp_ref, w_ref, b_ref, out_ref):
    dis = _dis_from(degp_ref[...])
    sblk = s_ref[...]
    hcat = jnp.concatenate([sblk[i] for i in range(NQ)], axis=1)
    h = jnp.maximum(hcat * dis + b_ref[...], 0.0)
    o = jnp.dot(h, w_ref[...], preferred_element_type=jnp.float32) * dis
    for qi in range(NQ):
        out_ref[qi] = o[:, qi * 128:(qi + 1) * 128]


def _tc3_body(s_ref, degp_ref, b_ref, wfc_ref, bfc_ref, out_ref):
    dis = _dis_from(degp_ref[...])
    sblk = s_ref[...]
    hcat = jnp.concatenate([sblk[i] for i in range(NQ)], axis=1)
    h = jnp.maximum(hcat * dis + b_ref[...], 0.0)
    lg = jnp.dot(h, wfc_ref[...], preferred_element_type=jnp.float32) + bfc_ref[...]
    m = jnp.max(lg, axis=1, keepdims=True)
    lse = jnp.log(jnp.sum(jnp.exp(lg - m), axis=1, keepdims=True)) + m
    out_ref[...] = lg - lse


_BM = 1024


def _tc1(x_p, W1, degp):
    return pl.pallas_call(
        _tc1_body,
        grid=(NP // _BM, NQ),
        in_specs=[
            pl.BlockSpec((_BM, D_IN), lambda i, j: (i, 0)),
            pl.BlockSpec((D_IN, 128), lambda i, j: (0, j)),
            pl.BlockSpec((NC, _BM, 16), lambda i, j: (0, i, 0)),
        ],
        out_specs=pl.BlockSpec((1, _BM, 128), lambda i, j: (j, i, 0)),
        out_shape=jax.ShapeDtypeStruct((NQ, NP, 128), jnp.float32),
    )(x_p, W1, degp)


def _tc2(s1, degp, W2, b1):
    return pl.pallas_call(
        _tc2_body,
        grid=(NP // _BM,),
        in_specs=[
            pl.BlockSpec((NQ, _BM, 128), lambda i: (0, i, 0)),
            pl.BlockSpec((NC, _BM, 16), lambda i: (0, i, 0)),
            pl.BlockSpec((D_HID, D_HID), lambda i: (0, 0)),
            pl.BlockSpec((1, D_HID), lambda i: (0, 0)),
        ],
        out_specs=pl.BlockSpec((NQ, _BM, 128), lambda i: (0, i, 0)),
        out_shape=jax.ShapeDtypeStruct((NQ, NP, 128), jnp.float32),
    )(s1, degp, W2, b1)


def _tc3(s2, degp, b2, Wfc, bfc):
    return pl.pallas_call(
        _tc3_body,
        grid=(NP // _BM,),
        in_specs=[
            pl.BlockSpec((NQ, _BM, 128), lambda i: (0, i, 0)),
            pl.BlockSpec((NC, _BM, 16), lambda i: (0, i, 0)),
            pl.BlockSpec((1, D_HID), lambda i: (0, 0)),
            pl.BlockSpec((D_HID, 64), lambda i: (0, 0)),
            pl.BlockSpec((1, 64), lambda i: (0, 0)),
        ],
        out_specs=pl.BlockSpec((_BM, 64), lambda i: (i, 0)),
        out_shape=jax.ShapeDtypeStruct((NP, 64), jnp.float32),
    )(s2, degp, b2, Wfc, bfc)


def kernel(x, edge_index, W1, b1, W2, b2, Wfc, bfc):
    src = edge_index[0].astype(jnp.int32)
    dst = edge_index[1].astype(jnp.int32)
    pad = EP - E
    src_p = jnp.concatenate([src, jnp.zeros((pad,), jnp.int32)])
    dst_p = jnp.concatenate([dst, jnp.full((pad,), N, jnp.int32)])
    src_l = src_p.reshape(NS, LCH, C)
    dst_l = dst_p.reshape(NS, LCH, C)
    dst_d = dst_p.reshape(NC, NS, DCH, C)
    ones16 = jnp.ones((C, 16), jnp.float32)
    zeros16 = jnp.zeros((RT, 16), jnp.float32)
    x_p = jnp.pad(x, ((0, NP - N), (0, 0)))

    degp = _deg_counts(dst_d, ones16, zeros16)
    g1 = _tc1(x_p, W1, degp)
    s1 = _gcn_aggregate(g1.reshape(NQ * NP, 128), src_l, dst_l)
    g2 = _tc2(s1, degp, W2, b1.reshape(1, -1))
    s2 = _gcn_aggregate(g2.reshape(NQ * NP, 128), src_l, dst_l)
    out = _tc3(s2, degp, b2.reshape(1, -1), Wfc, bfc.reshape(1, -1))
    return out[:N]
